# baseline (speedup 1.0000x reference)
import functools

import jax
import jax.numpy as jnp
from jax import lax
from jax.experimental import pallas as pl
from jax.experimental.pallas import tpu as pltpu
from jax.experimental.pallas import tpu_sc as plsc

_C = 256
_M = 128
_HW = 512 * 512
_K = 32
_D = _HW // _K
_NW = 32
_ROWS_OUT = _M * _K
_ROWS_PER_TILE = _ROWS_OUT // _NW
_CHUNK = 8
_NCHUNK = _ROWS_PER_TILE // _CHUNK
_L = 16


def _sc_body(table_hbm, cm_hbm, out_hbm, cm_v, idx_v, buf, sem):
    wid = lax.axis_index("s") * 2 + lax.axis_index("c")
    base = wid * _ROWS_PER_TILE

    iota = lax.broadcasted_iota(jnp.int32, (_L,), 0)
    for v in range(_ROWS_PER_TILE // _L):
        r16 = base + v * _L + iota
        idx_v[pl.ds(v * _L, _L)] = lax.shift_right_logical(r16, 5)
    pltpu.async_copy(cm_hbm.at[idx_v], cm_v, sem).wait()
    for v in range(_ROWS_PER_TILE // _L):
        r16 = base + v * _L + iota
        off = jnp.bitwise_and(r16, _K - 1)
        idx_v[pl.ds(v * _L, _L)] = cm_v[pl.ds(v * _L, _L)] * _K + off

    def chunk_body(c, carry):
        row0 = c * _CHUNK
        pltpu.async_copy(
            table_hbm.at[idx_v.at[pl.ds(row0, _CHUNK)]], buf, sem
        ).wait()
        pltpu.sync_copy(buf, out_hbm.at[pl.ds(base + row0, _CHUNK)])
        return carry

    lax.fori_loop(0, _NCHUNK, chunk_body, 0)


@jax.jit
def _sc_gather(table, channel_map):
    mesh = plsc.VectorSubcoreMesh(core_axis_name="c", subcore_axis_name="s")
    return pl.kernel(
        _sc_body,
        mesh=mesh,
        out_type=jax.ShapeDtypeStruct((_ROWS_OUT, _D), jnp.float32),
        scratch_types=[
            pltpu.VMEM((_ROWS_PER_TILE,), jnp.int32),
            pltpu.VMEM((_ROWS_PER_TILE,), jnp.int32),
            pltpu.VMEM((_CHUNK, _D), jnp.float32),
            pltpu.SemaphoreType.DMA,
        ],
    )(table, channel_map)


def kernel(model, channel_map):
    c, h, w = model.shape
    table = model.reshape(c * _K, (h * w) // _K)
    out = _sc_gather(table, channel_map.astype(jnp.int32))
    return out.reshape(channel_map.shape[0], h, w)
